# Initial kernel scaffold; baseline (speedup 1.0000x reference)
import jax
import jax.numpy as jnp
from jax.experimental import pallas as pl


def kernel(probs, y):
    raise NotImplementedError("write your pallas kernel here")


# speedup vs baseline: 2.0765x; 2.0765x over previous
import jax
import jax.numpy as jnp
from jax.experimental import pallas as pl
from jax.experimental.pallas import tpu as pltpu

_B = 16384
_C = 1000
_BR = 512


def _brier_body(y_ref, p_ref, out_ref):
    i = pl.program_id(0)
    p = p_ref[...]
    yb = y_ref[...]
    col = jax.lax.broadcasted_iota(jnp.int32, (_BR, _C), 1)
    hit = col == yb
    partial = jnp.sum(p * p) - 2.0 * jnp.sum(jnp.where(hit, p, 0.0))

    @pl.when(i == 0)
    def _init():
        out_ref[0, 0] = 0.0

    out_ref[0, 0] += partial

    @pl.when(i == pl.num_programs(0) - 1)
    def _fin():
        out_ref[0, 0] = (out_ref[0, 0] + jnp.float32(_B)) / jnp.float32(_B)


def kernel(probs, y):
    y2 = y.astype(jnp.int32).reshape(_B, 1)
    grid = _B // _BR
    out = pl.pallas_call(
        _brier_body,
        grid=(grid,),
        in_specs=[
            pl.BlockSpec((_BR, 1), lambda i: (i, 0)),
            pl.BlockSpec((_BR, _C), lambda i: (i, 0)),
        ],
        out_specs=pl.BlockSpec(
            (1, 1), lambda i: (0, 0), memory_space=pltpu.SMEM
        ),
        out_shape=jax.ShapeDtypeStruct((1, 1), jnp.float32),
    )(y2, probs)
    return out[0, 0]


# speedup vs baseline: 2.3328x; 1.1234x over previous
import jax
import jax.numpy as jnp
from jax.experimental import pallas as pl
from jax.experimental.pallas import tpu as pltpu

_B = 16384
_C = 1000
_BR = 2048


def _brier_body(y_ref, p_ref, out_ref):
    i = pl.program_id(0)
    p = p_ref[...]
    yb = y_ref[...]
    col = jax.lax.broadcasted_iota(jnp.int32, (_BR, _C), 1)
    hit = col == yb
    partial = jnp.sum(p * p) - 2.0 * jnp.sum(jnp.where(hit, p, 0.0))

    @pl.when(i == 0)
    def _init():
        out_ref[0, 0] = 0.0

    out_ref[0, 0] += partial

    @pl.when(i == pl.num_programs(0) - 1)
    def _fin():
        out_ref[0, 0] = (out_ref[0, 0] + jnp.float32(_B)) / jnp.float32(_B)


def kernel(probs, y):
    y2 = y.astype(jnp.int32).reshape(_B, 1)
    grid = _B // _BR
    out = pl.pallas_call(
        _brier_body,
        grid=(grid,),
        in_specs=[
            pl.BlockSpec((_BR, 1), lambda i: (i, 0)),
            pl.BlockSpec((_BR, _C), lambda i: (i, 0)),
        ],
        out_specs=pl.BlockSpec(
            (1, 1), lambda i: (0, 0), memory_space=pltpu.SMEM
        ),
        out_shape=jax.ShapeDtypeStruct((1, 1), jnp.float32),
    )(y2, probs)
    return out[0, 0]
